# baseline (speedup 1.0000x reference)
import functools

import jax
import jax.numpy as jnp
from jax import lax
from jax.experimental import pallas as pl
from jax.experimental.pallas import tpu as pltpu
from jax.experimental.pallas import tpu_sc as plsc

H = 512
W = 512
P = H * W
BC = 192
G = 4
NCORES = 2
NSUB = 16
GROUPS_PER_CORE = BC // (NCORES * G)
PT = P // NSUB
CH = 1024
NCHUNK = PT // CH
IROWS = 2 * CH // 128
NFIRE = IROWS
ZROWS = 256


def _corner_kernel(smx_ref, smy_ref, base_ref, w_ref):
    sx = smx_ref[...]
    sy = smy_ref[...]
    x0 = jnp.floor(sx)
    y0 = jnp.floor(sy)
    x0i = jnp.clip(x0.astype(jnp.int32), 0, W - 2)
    y0i = jnp.clip(y0.astype(jnp.int32), 0, H - 2)
    fx = sx - x0i.astype(jnp.float32)
    fy = sy - y0i.astype(jnp.float32)
    base_ref[...] = y0i * W + x0i
    w_ref[0] = (1.0 - fx) * (1.0 - fy)
    w_ref[1] = fx * (1.0 - fy)
    w_ref[2] = (1.0 - fx) * fy
    w_ref[3] = fx * fy


def _compute_corners(smx, smy):
    return pl.pallas_call(
        _corner_kernel,
        out_shape=[
            jax.ShapeDtypeStruct((H, W), jnp.int32),
            jax.ShapeDtypeStruct((4, H, W), jnp.float32),
        ],
    )(smx, smy)


def _sc_body(
    x_hbm,
    base_hbm,
    w_hbm,
    out_hbm,
    acc,
    xb,
    wb,
    bflat,
    idxq,
    vals,
    wstage,
    zb,
    sem,
    sem2,
    semz,
    semo,
    semb,
):
    cid = lax.axis_index("c")
    sid = lax.axis_index("s")
    iota = lax.iota(jnp.int32, 16)
    half = (iota >> 2) & 1
    colv = iota & 7
    civ = iota & 3
    unit2 = iota >> 3
    zero16 = jnp.zeros((16,), jnp.float32)
    base_t = sid * PT

    def z_init(i, c):
        zb[i, pl.ds(0, 8)] = jnp.zeros((8,), jnp.float32)
        return c

    def z_init16(i, c):
        plsc.store_scatter(zb, [i * 2 + unit2, colv], zero16)
        return c

    lax.fori_loop(0, ZROWS // 2, z_init16, 0)
    for z in range(PT // 2 // ZROWS):
        pltpu.sync_copy(
            zb, acc.at[pl.ds(pl.multiple_of(base_t // 2 + z * ZROWS, ZROWS), ZROWS)]
        )

    def zb2_init(i, c):
        rowv = CH + i * 2 + unit2
        for l in range(2):
            plsc.store_scatter(vals.at[l], [rowv, 4 + civ], zero16)
        return c

    lax.fori_loop(0, CH // 2, zb2_init, 0)
    plsc.subcore_barrier()

    def group_body(g, carry):
        pg = cid * (GROUPS_PER_CORE * G) + g * G

        def _bf(cc):
            base = pl.multiple_of(base_t + cc * CH, CH)
            return pltpu.make_async_copy(
                base_hbm.at[pl.ds(base, CH)], bflat.at[cc & 1], semb
            )

        _bf(0).start()

        def chunk_body(cc, carry2):
            base = pl.multiple_of(base_t + cc * CH, CH)
            ccp = cc & 1
            for gp in range(G):
                pltpu.async_copy(x_hbm.at[pg + gp, pl.ds(base, CH)], xb.at[gp], sem2)
            for k in range(4):
                pltpu.async_copy(w_hbm.at[k, pl.ds(base, CH)], wb.at[k], sem2)
            _bf(cc).wait()

            @pl.when(cc < NCHUNK - 1)
            def _prefetch_bf():
                _bf(cc + 1).start()

            for l in range(2):
                lw = l * W

                def d_body(j, c3, l=l):
                    pltpu.make_async_copy(
                        vals.at[l, pl.ds(pl.multiple_of(j * 128, 128), 128)],
                        acc.at[idxq.at[l, j]],
                        sem,
                    ).wait()
                    return c3

                @pl.when(cc > 0)
                def _drain():
                    lax.fori_loop(0, NFIRE, d_body, 0)

                def i_body(r, c3, l=l, lw=lw, ccp=ccp):
                    for v in range(8):
                        s = pl.ds(pl.multiple_of(v * 16, 16), 16)
                        sb = pl.ds(pl.multiple_of(r * 128 + v * 16, 16), 16)
                        bv = bflat[ccp, sb] + lw
                        idxq[l, r, s] = bv >> 1
                        idxq[l, (CH // 128) + r, s] = (bv + 1) >> 1
                    return c3

                lax.fori_loop(0, CH // 128, i_body, 0)

                if l == 0:
                    for gp in range(G):
                        pltpu.make_async_copy(
                            x_hbm.at[pg + gp, pl.ds(base, CH)], xb.at[gp], sem2
                        ).wait()
                    for k in range(4):
                        pltpu.make_async_copy(
                            w_hbm.at[k, pl.ds(base, CH)], wb.at[k], sem2
                        ).wait()

                def v_body(i, c3, l=l, lw=lw, ccp=ccp):
                    for u in range(2):
                        ii = i * 2 + u
                        s = pl.ds(pl.multiple_of(ii * 16, 16), 16)
                        pix16 = ii * 16 + iota
                        even = ((bflat[ccp, s] + lw) & 1) == 0
                        wa16 = wb[2 * l, s]
                        wb16 = wb[2 * l + 1, s]
                        rowA = pix16
                        rowB = CH + pix16
                        for gp in range(G):
                            xv = xb[gp, s]
                            pa = xv * wa16
                            pb = xv * wb16
                            cg = jnp.full((16,), gp, jnp.int32)
                            plsc.store_scatter(
                                vals.at[l], [rowA, cg], jnp.where(even, pa, 0.0)
                            )
                            plsc.store_scatter(
                                vals.at[l], [rowA, cg + 4], jnp.where(even, pb, pa)
                            )
                            plsc.store_scatter(
                                vals.at[l], [rowB, cg], jnp.where(even, 0.0, pb)
                            )
                    return c3

                lax.fori_loop(0, CH // 32, v_body, 0)

                def f_body(j, c3, l=l):
                    pltpu.async_copy(
                        vals.at[l, pl.ds(pl.multiple_of(j * 128, 128), 128)],
                        acc.at[idxq.at[l, j]],
                        sem,
                        add=True,
                    )
                    return c3

                lax.fori_loop(0, NFIRE, f_body, 0)
            return carry2

        lax.fori_loop(0, NCHUNK, chunk_body, 0)

        for l in range(2):
            def d_body(j, c3, l=l):
                pltpu.make_async_copy(
                    vals.at[l, pl.ds(pl.multiple_of(j * 128, 128), 128)],
                    acc.at[idxq.at[l, j]],
                    sem,
                ).wait()
                return c3

            lax.fori_loop(0, NFIRE, d_body, 0)
        plsc.subcore_barrier()

        def _stage(cc):
            r0 = pl.multiple_of(base_t // 2 + cc * (CH // 2), CH // 2)
            return pltpu.make_async_copy(
                acc.at[pl.ds(r0, CH // 2)], vals.at[cc & 1, pl.ds(0, CH // 2)], sem2
            )

        def _zero(cc, z):
            r0 = pl.multiple_of(base_t // 2 + cc * (CH // 2) + z * ZROWS, ZROWS)
            return pltpu.make_async_copy(zb, acc.at[pl.ds(r0, ZROWS)], semz)

        def _out(cc, gp):
            c0 = pl.multiple_of(base_t + cc * CH, CH)
            return pltpu.make_async_copy(
                wstage.at[gp], out_hbm.at[pg + gp, pl.ds(c0, CH)], semo
            )

        _stage(0).start()

        def wb_body(cc, carry2):
            _stage(cc).wait()

            @pl.when(cc < NCHUNK - 1)
            def _prefetch():
                _stage(cc + 1).start()

            for z in range(CH // 2 // ZROWS):
                _zero(cc, z).start()

            @pl.when(cc > 0)
            def _drain_out():
                for gp in range(G):
                    _out(cc - 1, gp).wait()

            bufv = jnp.full((16,), cc & 1, jnp.int32)

            def t_body(i, c3):
                for u in range(2):
                    ii = i * 2 + u
                    cv = ii * 16 + iota
                    rowv = cv >> 1
                    colb = (cv & 1) << 2
                    for gp in range(G):
                        v = plsc.load_gather(vals, [bufv, rowv, colb + gp])
                        wstage[gp, pl.ds(pl.multiple_of(ii * 16, 16), 16)] = v
                return c3

            lax.fori_loop(0, CH // 32, t_body, 0)
            for gp in range(G):
                _out(cc, gp).start()
            return carry2

        lax.fori_loop(0, NCHUNK, wb_body, 0)
        for gp in range(G):
            _out(NCHUNK - 1, gp).wait()

        def z_drain(cc, carry2):
            for z in range(CH // 2 // ZROWS):
                _zero(cc, z).wait()
            return carry2

        lax.fori_loop(0, NCHUNK, z_drain, 0)
        plsc.subcore_barrier()
        return carry

    lax.fori_loop(0, GROUPS_PER_CORE, group_body, 0)


def _sc_scatter(x2d, base_flat, w4):
    mesh = plsc.VectorSubcoreMesh(
        core_axis_name="c", subcore_axis_name="s", num_cores=NCORES, num_subcores=NSUB
    )
    fn = pl.kernel(
        _sc_body,
        out_type=jax.ShapeDtypeStruct((BC, P), jnp.float32),
        mesh=mesh,
        scratch_types=[
            pltpu.VMEM_SHARED((P // 2, 8), jnp.float32),
            pltpu.VMEM((G, CH), jnp.float32),
            pltpu.VMEM((4, CH), jnp.float32),
            pltpu.VMEM((2, CH), jnp.int32),
            pltpu.VMEM((2, IROWS, 128), jnp.int32),
            pltpu.VMEM((2, 2 * CH, 8), jnp.float32),
            pltpu.VMEM((G, CH), jnp.float32),
            pltpu.VMEM((ZROWS, 8), jnp.float32),
            pltpu.SemaphoreType.DMA,
            pltpu.SemaphoreType.DMA,
            pltpu.SemaphoreType.DMA,
            pltpu.SemaphoreType.DMA,
            pltpu.SemaphoreType.DMA,
        ],
        compiler_params=pltpu.CompilerParams(
            use_tc_tiling_on_sc=False, needs_layout_passes=False
        ),
    )
    return fn(x2d, base_flat, w4)


@jax.jit
def kernel(x, sample_map, output_shape):
    del output_shape
    B, C, Hin, Win = x.shape
    smx = sample_map[..., 0]
    smy = sample_map[..., 1]
    base_arr, w4 = _compute_corners(smx, smy)
    base_flat = base_arr.reshape(P)
    w4 = w4.reshape(4, P)
    x2d = x.reshape(B * C, Hin * Win)
    out = _sc_scatter(x2d, base_flat, w4)
    return out.reshape(B, C, H, W)


# speedup vs baseline: 1.0372x; 1.0372x over previous
import functools

import jax
import jax.numpy as jnp
from jax import lax
from jax.experimental import pallas as pl
from jax.experimental.pallas import tpu as pltpu
from jax.experimental.pallas import tpu_sc as plsc

H = 512
W = 512
P = H * W
BC = 192
G = 4
NCORES = 2
NSUB = 16
GROUPS_PER_CORE = BC // (NCORES * G)
PT = P // NSUB
CH = 1024
NCHUNK = PT // CH
IROWS = 2 * CH // 128
NFIRE = IROWS
ZROWS = 256


def _corner_kernel(smx_ref, smy_ref, base_ref, w_ref):
    sx = smx_ref[...]
    sy = smy_ref[...]
    x0 = jnp.floor(sx)
    y0 = jnp.floor(sy)
    x0i = jnp.clip(x0.astype(jnp.int32), 0, W - 2)
    y0i = jnp.clip(y0.astype(jnp.int32), 0, H - 2)
    fx = sx - x0i.astype(jnp.float32)
    fy = sy - y0i.astype(jnp.float32)
    base_ref[...] = y0i * W + x0i
    w_ref[0] = (1.0 - fx) * (1.0 - fy)
    w_ref[1] = fx * (1.0 - fy)
    w_ref[2] = (1.0 - fx) * fy
    w_ref[3] = fx * fy


def _compute_corners(smx, smy):
    return pl.pallas_call(
        _corner_kernel,
        out_shape=[
            jax.ShapeDtypeStruct((H, W), jnp.int32),
            jax.ShapeDtypeStruct((4, H, W), jnp.float32),
        ],
    )(smx, smy)


def _sc_body(
    x_hbm,
    base_hbm,
    w_hbm,
    out_hbm,
    acc,
    xb,
    wb,
    bflat,
    idxq,
    vals,
    wstage,
    zb,
    sem,
    sem2,
    semz,
    semo,
):
    cid = lax.axis_index("c")
    sid = lax.axis_index("s")
    iota = lax.iota(jnp.int32, 16)
    half = (iota >> 2) & 1
    colv = iota & 7
    civ = iota & 3
    unit2 = iota >> 3
    zero16 = jnp.zeros((16,), jnp.float32)
    base_t = sid * PT

    def z_init(i, c):
        zb[i, pl.ds(0, 8)] = jnp.zeros((8,), jnp.float32)
        return c

    def z_init16(i, c):
        plsc.store_scatter(zb, [i * 2 + unit2, colv], zero16)
        return c

    lax.fori_loop(0, ZROWS // 2, z_init16, 0)
    for z in range(PT // 2 // ZROWS):
        pltpu.sync_copy(
            zb, acc.at[pl.ds(pl.multiple_of(base_t // 2 + z * ZROWS, ZROWS), ZROWS)]
        )

    def zb2_init(i, c):
        rowv = CH + i * 2 + unit2
        for l in range(2):
            plsc.store_scatter(vals.at[l], [rowv, 4 + civ], zero16)
        return c

    lax.fori_loop(0, CH // 2, zb2_init, 0)
    plsc.subcore_barrier()

    def group_body(g, carry):
        pg = cid * (GROUPS_PER_CORE * G) + g * G

        def chunk_body(cc, carry2):
            base = pl.multiple_of(base_t + cc * CH, CH)
            for gp in range(G):
                pltpu.async_copy(x_hbm.at[pg + gp, pl.ds(base, CH)], xb.at[gp], sem2)
            for k in range(4):
                pltpu.async_copy(w_hbm.at[k, pl.ds(base, CH)], wb.at[k], sem2)
            pltpu.sync_copy(base_hbm.at[pl.ds(base, CH)], bflat)

            for l in range(2):
                lw = l * W

                def d_body(j, c3, l=l):
                    pltpu.make_async_copy(
                        vals.at[l, pl.ds(pl.multiple_of(j * 128, 128), 128)],
                        acc.at[idxq.at[l, j]],
                        sem,
                    ).wait()
                    return c3

                @pl.when(cc > 0)
                def _drain():
                    lax.fori_loop(0, NFIRE, d_body, 0)

                def i_body(r, c3, l=l, lw=lw):
                    for v in range(8):
                        s = pl.ds(pl.multiple_of(v * 16, 16), 16)
                        bv = bflat[pl.ds(pl.multiple_of(r * 128 + v * 16, 16), 16)] + lw
                        idxq[l, r, s] = bv >> 1
                        idxq[l, (CH // 128) + r, s] = (bv + 1) >> 1
                    return c3

                lax.fori_loop(0, CH // 128, i_body, 0)

                if l == 0:
                    for gp in range(G):
                        pltpu.make_async_copy(
                            x_hbm.at[pg + gp, pl.ds(base, CH)], xb.at[gp], sem2
                        ).wait()
                    for k in range(4):
                        pltpu.make_async_copy(
                            w_hbm.at[k, pl.ds(base, CH)], wb.at[k], sem2
                        ).wait()

                def v_body(i, c3, l=l, lw=lw):
                    for u in range(2):
                        ii = i * 2 + u
                        s = pl.ds(pl.multiple_of(ii * 16, 16), 16)
                        pix16 = ii * 16 + iota
                        even = ((bflat[s] + lw) & 1) == 0
                        wa16 = wb[2 * l, s]
                        wb16 = wb[2 * l + 1, s]
                        rowA = pix16
                        rowB = CH + pix16
                        for gp in range(G):
                            xv = xb[gp, s]
                            pa = xv * wa16
                            pb = xv * wb16
                            cg = jnp.full((16,), gp, jnp.int32)
                            plsc.store_scatter(
                                vals.at[l], [rowA, cg], jnp.where(even, pa, 0.0)
                            )
                            plsc.store_scatter(
                                vals.at[l], [rowA, cg + 4], jnp.where(even, pb, pa)
                            )
                            plsc.store_scatter(
                                vals.at[l], [rowB, cg], jnp.where(even, 0.0, pb)
                            )
                    return c3

                lax.fori_loop(0, CH // 32, v_body, 0)

                def f_body(j, c3, l=l):
                    pltpu.async_copy(
                        vals.at[l, pl.ds(pl.multiple_of(j * 128, 128), 128)],
                        acc.at[idxq.at[l, j]],
                        sem,
                        add=True,
                    )
                    return c3

                lax.fori_loop(0, NFIRE, f_body, 0)
            return carry2

        lax.fori_loop(0, NCHUNK, chunk_body, 0)

        for l in range(2):
            def d_body(j, c3, l=l):
                pltpu.make_async_copy(
                    vals.at[l, pl.ds(pl.multiple_of(j * 128, 128), 128)],
                    acc.at[idxq.at[l, j]],
                    sem,
                ).wait()
                return c3

            lax.fori_loop(0, NFIRE, d_body, 0)
        plsc.subcore_barrier()

        def _stage(cc):
            r0 = pl.multiple_of(base_t // 2 + cc * (CH // 2), CH // 2)
            return pltpu.make_async_copy(
                acc.at[pl.ds(r0, CH // 2)], vals.at[cc & 1, pl.ds(0, CH // 2)], sem2
            )

        def _zero(cc, z):
            r0 = pl.multiple_of(base_t // 2 + cc * (CH // 2) + z * ZROWS, ZROWS)
            return pltpu.make_async_copy(zb, acc.at[pl.ds(r0, ZROWS)], semz)

        def _out(cc, gp):
            c0 = pl.multiple_of(base_t + cc * CH, CH)
            return pltpu.make_async_copy(
                wstage.at[gp], out_hbm.at[pg + gp, pl.ds(c0, CH)], semo
            )

        _stage(0).start()

        def wb_body(cc, carry2):
            _stage(cc).wait()

            @pl.when(cc < NCHUNK - 1)
            def _prefetch():
                _stage(cc + 1).start()

            for z in range(CH // 2 // ZROWS):
                _zero(cc, z).start()

            @pl.when(cc > 0)
            def _drain_out():
                for gp in range(G):
                    _out(cc - 1, gp).wait()

            bufv = jnp.full((16,), cc & 1, jnp.int32)

            def t_body(i, c3):
                for u in range(2):
                    ii = i * 2 + u
                    cv = ii * 16 + iota
                    rowv = cv >> 1
                    colb = (cv & 1) << 2
                    for gp in range(G):
                        v = plsc.load_gather(vals, [bufv, rowv, colb + gp])
                        wstage[gp, pl.ds(pl.multiple_of(ii * 16, 16), 16)] = v
                return c3

            lax.fori_loop(0, CH // 32, t_body, 0)
            for gp in range(G):
                _out(cc, gp).start()
            return carry2

        lax.fori_loop(0, NCHUNK, wb_body, 0)
        for gp in range(G):
            _out(NCHUNK - 1, gp).wait()

        def z_drain(cc, carry2):
            for z in range(CH // 2 // ZROWS):
                _zero(cc, z).wait()
            return carry2

        lax.fori_loop(0, NCHUNK, z_drain, 0)
        plsc.subcore_barrier()
        return carry

    lax.fori_loop(0, GROUPS_PER_CORE, group_body, 0)


def _sc_scatter(x2d, base_flat, w4):
    mesh = plsc.VectorSubcoreMesh(
        core_axis_name="c", subcore_axis_name="s", num_cores=NCORES, num_subcores=NSUB
    )
    fn = pl.kernel(
        _sc_body,
        out_type=jax.ShapeDtypeStruct((BC, P), jnp.float32),
        mesh=mesh,
        scratch_types=[
            pltpu.VMEM_SHARED((P // 2, 8), jnp.float32),
            pltpu.VMEM((G, CH), jnp.float32),
            pltpu.VMEM((4, CH), jnp.float32),
            pltpu.VMEM((CH,), jnp.int32),
            pltpu.VMEM((2, IROWS, 128), jnp.int32),
            pltpu.VMEM((2, 2 * CH, 8), jnp.float32),
            pltpu.VMEM((G, CH), jnp.float32),
            pltpu.VMEM((ZROWS, 8), jnp.float32),
            pltpu.SemaphoreType.DMA,
            pltpu.SemaphoreType.DMA,
            pltpu.SemaphoreType.DMA,
            pltpu.SemaphoreType.DMA,
        ],
        compiler_params=pltpu.CompilerParams(
            use_tc_tiling_on_sc=False, needs_layout_passes=False
        ),
    )
    return fn(x2d, base_flat, w4)


@jax.jit
def kernel(x, sample_map, output_shape):
    del output_shape
    B, C, Hin, Win = x.shape
    smx = sample_map[..., 0]
    smy = sample_map[..., 1]
    base_arr, w4 = _compute_corners(smx, smy)
    base_flat = base_arr.reshape(P)
    w4 = w4.reshape(4, P)
    x2d = x.reshape(B * C, Hin * Win)
    out = _sc_scatter(x2d, base_flat, w4)
    return out.reshape(B, C, H, W)
